# baseline (device time: 71708 ns/iter reference)
import jax
import jax.numpy as jnp
from jax import lax
from jax.experimental import pallas as pl
from jax.experimental.pallas import tpu as pltpu

N_DEV = 8


def kernel(A, B):
    m, _ = A.shape
    _, n = B.shape
    chunk = m // N_DEV
    half = chunk // 2

    def body(a_ref, b_ref, out_ref, comm_r, comm_l, a16, b16,
             send_r, recv_r, send_l, recv_l):
        d = lax.axis_index("i")
        left = lax.rem(d + N_DEV - 1, N_DEV)
        right = lax.rem(d + 1, N_DEV)

        a16[...] = a_ref[...].astype(jnp.bfloat16)
        b16[...] = b_ref[...].astype(jnp.bfloat16)

        barrier_sem = pltpu.get_barrier_semaphore()
        for nbr in (left, right):
            pl.semaphore_signal(
                barrier_sem, inc=1,
                device_id=(nbr,), device_id_type=pl.DeviceIdType.MESH,
            )
        pl.semaphore_wait(barrier_sem, 2)

        def partial_rows(row0):
            return jnp.dot(
                a16[pl.ds(row0, half), :], b16[...],
                preferred_element_type=jnp.float32,
            )

        c0r = lax.rem(d + N_DEV - 1, N_DEV)
        c0l = lax.rem(d + 1, N_DEV)
        comm_r[N_DEV - 1] = partial_rows(c0r * chunk).astype(jnp.bfloat16)
        comm_l[N_DEV - 1] = partial_rows(c0l * chunk + half).astype(jnp.bfloat16)

        for s in range(N_DEV - 1):
            src = N_DEV - 1 if s == 0 else s - 1
            rdma_r = pltpu.make_async_remote_copy(
                src_ref=comm_r.at[src], dst_ref=comm_r.at[s],
                send_sem=send_r.at[s], recv_sem=recv_r.at[s],
                device_id=(right,), device_id_type=pl.DeviceIdType.MESH,
            )
            rdma_l = pltpu.make_async_remote_copy(
                src_ref=comm_l.at[src], dst_ref=comm_l.at[s],
                send_sem=send_l.at[s], recv_sem=recv_l.at[s],
                device_id=(left,), device_id_type=pl.DeviceIdType.MESH,
            )
            rdma_r.start()
            rdma_l.start()
            cr = lax.rem(d + 2 * N_DEV - s - 2, N_DEV)
            cl = lax.rem(d + s + 2, N_DEV)
            pr = partial_rows(cr * chunk)
            pl_ = partial_rows(cl * chunk + half)
            rdma_r.wait()
            rdma_l.wait()
            if s < N_DEV - 2:
                comm_r[s] = (comm_r[s].astype(jnp.float32) + pr).astype(jnp.bfloat16)
                comm_l[s] = (comm_l[s].astype(jnp.float32) + pl_).astype(jnp.bfloat16)
            else:
                out_ref[:half, :] = comm_r[s].astype(jnp.float32) + pr
                out_ref[half:, :] = comm_l[s].astype(jnp.float32) + pl_

    return pl.pallas_call(
        body,
        out_shape=jax.ShapeDtypeStruct((chunk, n), jnp.float32),
        in_specs=[
            pl.BlockSpec(memory_space=pltpu.VMEM),
            pl.BlockSpec(memory_space=pltpu.VMEM),
        ],
        out_specs=pl.BlockSpec(memory_space=pltpu.VMEM),
        scratch_shapes=[
            pltpu.VMEM((N_DEV, half, n), jnp.bfloat16),
            pltpu.VMEM((N_DEV, half, n), jnp.bfloat16),
            pltpu.VMEM((m, A.shape[1]), jnp.bfloat16),
            pltpu.VMEM((B.shape[0], n), jnp.bfloat16),
            pltpu.SemaphoreType.DMA((N_DEV - 1,)),
            pltpu.SemaphoreType.DMA((N_DEV - 1,)),
            pltpu.SemaphoreType.DMA((N_DEV - 1,)),
            pltpu.SemaphoreType.DMA((N_DEV - 1,)),
        ],
        compiler_params=pltpu.CompilerParams(collective_id=0),
    )(A, B)


# device time: 56827 ns/iter; 1.2619x vs baseline; 1.2619x over previous
import jax
import jax.numpy as jnp
from jax import lax
from jax.experimental import pallas as pl
from jax.experimental.pallas import tpu as pltpu

N_DEV = 8
N_HOP = N_DEV - 1


def kernel(A, B):
    m, _ = A.shape
    _, n = B.shape
    chunk = m // N_DEV
    half = chunk // 2
    sub = half // 2

    def body(a_ref, b_ref, out_ref,
             c_r0, c_r1, c_l0, c_l1, a16, b16,
             send_r0, recv_r0, send_r1, recv_r1,
             send_l0, recv_l0, send_l1, recv_l1):
        d = lax.axis_index("i")
        left = lax.rem(d + N_DEV - 1, N_DEV)
        right = lax.rem(d + 1, N_DEV)

        a16[...] = a_ref[...].astype(jnp.bfloat16)
        b16[...] = b_ref[...].astype(jnp.bfloat16)

        barrier_sem = pltpu.get_barrier_semaphore()
        for nbr in (left, right):
            pl.semaphore_signal(
                barrier_sem, inc=1,
                device_id=(nbr,), device_id_type=pl.DeviceIdType.MESH,
            )
        pl.semaphore_wait(barrier_sem, 2)

        def dot_rows(row0):
            return jnp.dot(
                a16[pl.ds(row0, half), :], b16[...],
                preferred_element_type=jnp.float32,
            )

        streams = (
            (c_r0, send_r0, recv_r0, right, 0),
            (c_l0, send_l0, recv_l0, left, 2 * sub),
            (c_r1, send_r1, recv_r1, right, sub),
            (c_l1, send_l1, recv_l1, left, 3 * sub),
        )

        rdmas = {}

        def start_hop(q, s):
            comm, ssem, rsem, nbr, _ = streams[q]
            src = N_HOP if s == 0 else s - 1
            rd = pltpu.make_async_remote_copy(
                src_ref=comm.at[src], dst_ref=comm.at[s],
                send_sem=ssem.at[s], recv_sem=rsem.at[s],
                device_id=(nbr,), device_id_type=pl.DeviceIdType.MESH,
            )
            rdmas[(q, s)] = rd
            rd.start()

        pr = dot_rows(lax.rem(d + N_DEV - 1, N_DEV) * chunk)
        pl_ = dot_rows(lax.rem(d + 1, N_DEV) * chunk + half)
        c_r0[N_HOP] = pr[:sub].astype(jnp.bfloat16)
        c_r1[N_HOP] = pr[sub:].astype(jnp.bfloat16)
        c_l0[N_HOP] = pl_[:sub].astype(jnp.bfloat16)
        c_l1[N_HOP] = pl_[sub:].astype(jnp.bfloat16)
        for q in range(4):
            start_hop(q, 0)

        for s in range(N_HOP):
            cr = lax.rem(d + 2 * N_DEV - s - 2, N_DEV)
            cl = lax.rem(d + s + 2, N_DEV)
            pr = dot_rows(cr * chunk)
            pl_ = dot_rows(cl * chunk + half)
            parts = (pr[:sub], pl_[:sub], pr[sub:], pl_[sub:])
            for q in (0, 1, 2, 3):
                comm, _, _, _, out_row = streams[q]
                rdmas[(q, s)].wait_recv()
                acc = comm[s].astype(jnp.float32) + parts[q]
                if s < N_HOP - 1:
                    comm[s] = acc.astype(jnp.bfloat16)
                    start_hop(q, s + 1)
                else:
                    out_ref[pl.ds(out_row, sub), :] = acc

        for q in range(4):
            for s in range(N_HOP):
                rdmas[(q, s)].wait_send()

    comm_shape = pltpu.VMEM((N_DEV, sub, n), jnp.bfloat16)
    hop_sems = pltpu.SemaphoreType.DMA((N_HOP,))
    return pl.pallas_call(
        body,
        out_shape=jax.ShapeDtypeStruct((chunk, n), jnp.float32),
        in_specs=[
            pl.BlockSpec(memory_space=pltpu.VMEM),
            pl.BlockSpec(memory_space=pltpu.VMEM),
        ],
        out_specs=pl.BlockSpec(memory_space=pltpu.VMEM),
        scratch_shapes=[
            comm_shape, comm_shape, comm_shape, comm_shape,
            pltpu.VMEM((m, A.shape[1]), jnp.bfloat16),
            pltpu.VMEM((B.shape[0], n), jnp.bfloat16),
            hop_sems, hop_sems, hop_sems, hop_sems,
            hop_sems, hop_sems, hop_sems, hop_sems,
        ],
        compiler_params=pltpu.CompilerParams(collective_id=0),
    )(A, B)


# device time: 55833 ns/iter; 1.2843x vs baseline; 1.0178x over previous
import jax
import jax.numpy as jnp
from jax import lax
from jax.experimental import pallas as pl
from jax.experimental.pallas import tpu as pltpu

N_DEV = 8
N_HOP = N_DEV - 1


def kernel(A, B):
    m, _ = A.shape
    _, n = B.shape
    chunk = m // N_DEV
    half = chunk // 2
    sub = half // 2

    def body(a_ref, b_ref, out_ref,
             c_r0, c_r1, c_l0, c_l1, b16,
             send_r0, recv_r0, send_r1, recv_r1,
             send_l0, recv_l0, send_l1, recv_l1):
        d = lax.axis_index("i")
        left = lax.rem(d + N_DEV - 1, N_DEV)
        right = lax.rem(d + 1, N_DEV)

        b16[...] = b_ref[...].astype(jnp.bfloat16)

        def dot_rows(row0):
            return jnp.dot(
                a_ref[pl.ds(row0, half), :].astype(jnp.bfloat16), b16[...],
                preferred_element_type=jnp.float32,
            )

        streams = (
            (c_r0, send_r0, recv_r0, right, 0),
            (c_l0, send_l0, recv_l0, left, 2 * sub),
            (c_r1, send_r1, recv_r1, right, sub),
            (c_l1, send_l1, recv_l1, left, 3 * sub),
        )

        rdmas = {}

        def start_hop(q, s):
            comm, ssem, rsem, nbr, _ = streams[q]
            src = N_HOP if s == 0 else s - 1
            rd = pltpu.make_async_remote_copy(
                src_ref=comm.at[src], dst_ref=comm.at[s],
                send_sem=ssem.at[s], recv_sem=rsem.at[s],
                device_id=(nbr,), device_id_type=pl.DeviceIdType.MESH,
            )
            rdmas[(q, s)] = rd
            rd.start()

        pr = dot_rows(lax.rem(d + N_DEV - 1, N_DEV) * chunk)
        pl_ = dot_rows(lax.rem(d + 1, N_DEV) * chunk + half)
        c_r0[N_HOP] = pr[:sub].astype(jnp.bfloat16)
        c_r1[N_HOP] = pr[sub:].astype(jnp.bfloat16)
        c_l0[N_HOP] = pl_[:sub].astype(jnp.bfloat16)
        c_l1[N_HOP] = pl_[sub:].astype(jnp.bfloat16)

        barrier_sem = pltpu.get_barrier_semaphore()
        for nbr in (left, right):
            pl.semaphore_signal(
                barrier_sem, inc=1,
                device_id=(nbr,), device_id_type=pl.DeviceIdType.MESH,
            )
        pl.semaphore_wait(barrier_sem, 2)

        for q in range(4):
            start_hop(q, 0)

        for s in range(N_HOP):
            cr = lax.rem(d + 2 * N_DEV - s - 2, N_DEV)
            cl = lax.rem(d + s + 2, N_DEV)
            pr = dot_rows(cr * chunk)
            pl_ = dot_rows(cl * chunk + half)
            parts = (pr[:sub], pl_[:sub], pr[sub:], pl_[sub:])
            for q in (0, 1, 2, 3):
                comm, _, _, _, out_row = streams[q]
                rdmas[(q, s)].wait_recv()
                acc = comm[s].astype(jnp.float32) + parts[q]
                if s < N_HOP - 1:
                    comm[s] = acc.astype(jnp.bfloat16)
                    start_hop(q, s + 1)
                else:
                    out_ref[pl.ds(out_row, sub), :] = acc

        for q in range(4):
            for s in range(N_HOP):
                rdmas[(q, s)].wait_send()

    comm_shape = pltpu.VMEM((N_DEV, sub, n), jnp.bfloat16)
    hop_sems = pltpu.SemaphoreType.DMA((N_HOP,))
    return pl.pallas_call(
        body,
        out_shape=jax.ShapeDtypeStruct((chunk, n), jnp.float32),
        in_specs=[
            pl.BlockSpec(memory_space=pltpu.VMEM),
            pl.BlockSpec(memory_space=pltpu.VMEM),
        ],
        out_specs=pl.BlockSpec(memory_space=pltpu.VMEM),
        scratch_shapes=[
            comm_shape, comm_shape, comm_shape, comm_shape,
            pltpu.VMEM((B.shape[0], n), jnp.bfloat16),
            hop_sems, hop_sems, hop_sems, hop_sems,
            hop_sems, hop_sems, hop_sems, hop_sems,
        ],
        compiler_params=pltpu.CompilerParams(collective_id=0),
    )(A, B)
